# Initial kernel scaffold; baseline (speedup 1.0000x reference)
import jax
import jax.numpy as jnp
from jax.experimental import pallas as pl


def kernel(rt_cls_attn, rt, rt_centroids, shift_and_scale, anc_indices, pos_indices, neg_indices, W_in, b_in, W1, b1, W2, b2):
    raise NotImplementedError("write your pallas kernel here")


# speedup vs baseline: 1.1458x; 1.1458x over previous
import functools

import jax
import jax.numpy as jnp
from jax import lax
from jax.experimental import pallas as pl
from jax.experimental.pallas import tpu as pltpu

B = 64
N = 8192
C = 128
K = 256
T = 64
D_THRESH = 0.5
POWER_ITERS = 10


def _proj_body(rtk_ref, W_ref, b_ref, cent_ref, ss_ref, normP_ref, centu_ref):
    b = pl.program_id(0)
    rtk = rtk_ref[0]
    proj = jnp.dot(rtk, W_ref[...], preferred_element_type=jnp.float32)
    proj = proj + b_ref[...]
    nrm = jnp.sqrt(jnp.sum(proj * proj, axis=1, keepdims=True))
    normP_ref[0] = proj / (nrm + 1e-6)

    cent = cent_ref[0]
    scale = ss_ref[b, 3]
    ri = lax.broadcasted_iota(jnp.int32, (8, K), 0)
    shift = jnp.where(ri == 0, ss_ref[b, 0],
            jnp.where(ri == 1, ss_ref[b, 1],
            jnp.where(ri == 2, ss_ref[b, 2], 0.0)))
    centu_ref[0] = cent * scale + shift


def _run_proj(rt_k, cent_cm, shift_and_scale, W_in, b_in):
    return pl.pallas_call(
        _proj_body,
        grid=(B,),
        in_specs=[
            pl.BlockSpec((1, K, C), lambda b: (b, 0, 0)),
            pl.BlockSpec((C, C), lambda b: (0, 0)),
            pl.BlockSpec((1, C), lambda b: (0, 0)),
            pl.BlockSpec((1, 8, K), lambda b: (b, 0, 0)),
            pl.BlockSpec(memory_space=pltpu.SMEM),
        ],
        out_specs=[
            pl.BlockSpec((1, K, C), lambda b: (b, 0, 0)),
            pl.BlockSpec((1, 8, K), lambda b: (b, 0, 0)),
        ],
        out_shape=[
            jax.ShapeDtypeStruct((B, K, C), jnp.float32),
            jax.ShapeDtypeStruct((B, 8, K), jnp.float32),
        ],
    )(rt_k, W_in, b_in.reshape(1, C), cent_cm, shift_and_scale)


def _col(row, diag):
    return jnp.sum(jnp.where(diag, row, 0.0), axis=1, keepdims=True)


def _pdist(cm, diag):
    acc = jnp.zeros((K, K), jnp.float32)
    for c in range(3):
        xj = cm[c:c + 1, :]
        xi = _col(xj, diag)
        d = xi - xj
        acc = acc + d * d
    return jnp.sqrt(acc + 1e-12)


def _sgv_body(anc_i, pos_i, neg_i,
              ancP_ref, posP_ref, negP_ref,
              ancC_ref, posC_ref, negC_ref,
              W1_ref, b1_ref, w2t_ref, b2_ref,
              rer_ref, sc_ref):
    del anc_i, pos_i, neg_i
    ri = lax.broadcasted_iota(jnp.int32, (K, K), 0)
    ci = lax.broadcasted_iota(jnp.int32, (K, K), 1)
    diag = ri == ci

    anc_n = ancP_ref[0]
    d_anc = _pdist(ancC_ref[0], diag)

    li = lax.broadcasted_iota(jnp.int32, (1, 128), 1)
    rer_acc = jnp.zeros((1, 128), jnp.float32)
    sc_acc = jnp.zeros((1, 128), jnp.float32)

    for n, (nP_ref, nC_ref) in enumerate(((posP_ref, posC_ref),
                                          (negP_ref, negC_ref))):
        nn_n = nP_ref[0]
        d_nn = _pdist(nC_ref[0], diag)
        beta = (d_anc - d_nn) ** 2 * (1.0 / (D_THRESH * D_THRESH))
        geo = jnp.maximum(1.0 - beta, 0.0)
        featT = lax.dot_general(nn_n, anc_n, (((1,), (1,)), ((), ())),
                                preferred_element_type=jnp.float32)
        AT = jnp.where(diag, 0.0, geo * jnp.maximum(featT, 0.0))

        v = jnp.full((1, K), 1.0 / (K ** 0.5), jnp.float32)
        for _ in range(POWER_ITERS):
            u = jnp.dot(v, AT, preferred_element_type=jnp.float32)
            v = u / (jnp.sqrt(jnp.sum(u * u)) + 1e-8)
        w = jnp.dot(v, AT, preferred_element_type=jnp.float32)
        sc_val = jnp.sum(w * v)

        v_colv = _col(v, diag)
        gt = (v > v_colv).astype(jnp.float32)
        tie = ((v == v_colv) & (ci < ri)).astype(jnp.float32)
        r = jnp.sum(gt + tie, axis=1, keepdims=True)
        Q = (r == ci.astype(jnp.float32)).astype(jnp.float32)
        sorted_v = jnp.dot(v, Q, preferred_element_type=jnp.float32)

        h = jnp.maximum(
            jnp.dot(sorted_v, W1_ref[...], preferred_element_type=jnp.float32)
            + b1_ref[...], 0.0)
        o = jnp.sum(h * w2t_ref[...]) + b2_ref[0, 0]
        rer_acc = rer_acc + jnp.where(li == n, 1.0 / (1.0 + jnp.exp(-o)), 0.0)
        sc_acc = sc_acc + jnp.where(li == n, sc_val, 0.0)

    rer_ref[0] = rer_acc
    sc_ref[0] = sc_acc


def _run_sgv(normP, cent_u, anc_indices, pos_indices, neg_indices,
             W1, b1, W2, b2):
    grid_spec = pltpu.PrefetchScalarGridSpec(
        num_scalar_prefetch=3,
        grid=(T,),
        in_specs=[
            pl.BlockSpec((1, K, C), lambda t, a, p, n: (a[t], 0, 0)),
            pl.BlockSpec((1, K, C), lambda t, a, p, n: (p[t], 0, 0)),
            pl.BlockSpec((1, K, C), lambda t, a, p, n: (n[t], 0, 0)),
            pl.BlockSpec((1, 8, K), lambda t, a, p, n: (a[t], 0, 0)),
            pl.BlockSpec((1, 8, K), lambda t, a, p, n: (p[t], 0, 0)),
            pl.BlockSpec((1, 8, K), lambda t, a, p, n: (n[t], 0, 0)),
            pl.BlockSpec((K, K), lambda t, a, p, n: (0, 0)),
            pl.BlockSpec((1, K), lambda t, a, p, n: (0, 0)),
            pl.BlockSpec((1, K), lambda t, a, p, n: (0, 0)),
            pl.BlockSpec(memory_space=pltpu.SMEM),
        ],
        out_specs=[
            pl.BlockSpec((1, 1, 128), lambda t, a, p, n: (t, 0, 0)),
            pl.BlockSpec((1, 1, 128), lambda t, a, p, n: (t, 0, 0)),
        ],
    )
    rer_pad, sc_pad = pl.pallas_call(
        _sgv_body,
        grid_spec=grid_spec,
        out_shape=[
            jax.ShapeDtypeStruct((T, 1, 128), jnp.float32),
            jax.ShapeDtypeStruct((T, 1, 128), jnp.float32),
        ],
    )(anc_indices, pos_indices, neg_indices,
      normP, normP, normP, cent_u, cent_u, cent_u,
      W1, b1.reshape(1, K), W2.reshape(1, K), b2.reshape(1, 1))
    return rer_pad[:, 0, :2], sc_pad[:, 0, :2]


def kernel(rt_cls_attn, rt, rt_centroids, shift_and_scale,
           anc_indices, pos_indices, neg_indices,
           W_in, b_in, W1, b1, W2, b2):
    scores = rt_cls_attn[..., 0]
    _, topk_idx = lax.top_k(scores, K)
    topk_idx = jnp.sort(topk_idx, axis=-1)
    rt_k = jnp.take_along_axis(rt, topk_idx[..., None], axis=1)
    cent_k = jnp.take_along_axis(rt_centroids, topk_idx[..., None], axis=1)
    cent_cm = jnp.transpose(cent_k, (0, 2, 1))
    cent_cm = jnp.pad(cent_cm, ((0, 0), (0, 5), (0, 0)))

    normP, cent_u = _run_proj(rt_k, cent_cm, shift_and_scale, W_in, b_in)
    return _run_sgv(normP, cent_u, anc_indices, pos_indices, neg_indices,
                    W1, b1, W2, b2)


# speedup vs baseline: 1.9534x; 1.7048x over previous
import functools

import jax
import jax.numpy as jnp
from jax import lax
from jax.experimental import pallas as pl
from jax.experimental.pallas import tpu as pltpu

B = 64
N = 8192
C = 128
K = 256
T = 64
D_THRESH = 0.5
POWER_ITERS = 10


def _proj_body(rtk_ref, W_ref, b_ref, cent_ref, ss_ref, normP_ref, centu_ref):
    b = pl.program_id(0)
    rtk = rtk_ref[0]
    proj = jnp.dot(rtk, W_ref[...], preferred_element_type=jnp.float32)
    proj = proj + b_ref[...]
    nrm = jnp.sqrt(jnp.sum(proj * proj, axis=1, keepdims=True))
    normP_ref[0] = proj / (nrm + 1e-6)

    cent = cent_ref[0]
    scale = ss_ref[b, 3]
    ri = lax.broadcasted_iota(jnp.int32, (8, K), 0)
    shift = jnp.where(ri == 0, ss_ref[b, 0],
            jnp.where(ri == 1, ss_ref[b, 1],
            jnp.where(ri == 2, ss_ref[b, 2], 0.0)))
    centu_ref[0] = cent * scale + shift


def _run_proj(rt_k, cent_cm, shift_and_scale, W_in, b_in):
    return pl.pallas_call(
        _proj_body,
        grid=(B,),
        in_specs=[
            pl.BlockSpec((1, K, C), lambda b: (b, 0, 0)),
            pl.BlockSpec((C, C), lambda b: (0, 0)),
            pl.BlockSpec((1, C), lambda b: (0, 0)),
            pl.BlockSpec((1, 8, K), lambda b: (b, 0, 0)),
            pl.BlockSpec(memory_space=pltpu.SMEM),
        ],
        out_specs=[
            pl.BlockSpec((1, K, C), lambda b: (b, 0, 0)),
            pl.BlockSpec((1, 8, K), lambda b: (b, 0, 0)),
        ],
        out_shape=[
            jax.ShapeDtypeStruct((B, K, C), jnp.float32),
            jax.ShapeDtypeStruct((B, 8, K), jnp.float32),
        ],
    )(rt_k, W_in, b_in.reshape(1, C), cent_cm, shift_and_scale)


def _col(row, diag):
    return jnp.sum(jnp.where(diag, row, 0.0), axis=1, keepdims=True)


def _pdist(cm, diag):
    acc = jnp.zeros((K, K), jnp.float32)
    for c in range(3):
        xj = cm[c:c + 1, :]
        xi = _col(xj, diag)
        d = xi - xj
        acc = acc + d * d
    return jnp.sqrt(acc + 1e-12)


G = 4


def _sgv_body(anc_i, pos_i, neg_i, *refs):
    del anc_i, pos_i, neg_i
    P_refs = refs[0:3 * G]
    C_refs = refs[3 * G:6 * G]
    W1_ref, b1_ref, w2t_ref, b2_ref = refs[6 * G:6 * G + 4]
    rer_ref, sc_ref = refs[6 * G + 4:]

    ri = lax.broadcasted_iota(jnp.int32, (K, K), 0)
    ci = lax.broadcasted_iota(jnp.int32, (K, K), 1)
    diag = ri == ci

    ATs = []
    for g in range(G):
        anc_n = P_refs[3 * g][0]
        d_anc = _pdist(C_refs[3 * g][0], diag)
        for n in (1, 2):
            nn_n = P_refs[3 * g + n][0]
            d_nn = _pdist(C_refs[3 * g + n][0], diag)
            beta = (d_anc - d_nn) ** 2 * (1.0 / (D_THRESH * D_THRESH))
            geo = jnp.maximum(1.0 - beta, 0.0)
            featT = lax.dot_general(nn_n, anc_n, (((1,), (1,)), ((), ())),
                                    preferred_element_type=jnp.float32)
            ATs.append(jnp.where(diag, 0.0, geo * jnp.maximum(featT, 0.0)))

    vs = [jnp.full((1, K), 1.0 / (K ** 0.5), jnp.float32)] * (2 * G)
    for _ in range(POWER_ITERS):
        us = [jnp.dot(v, AT, preferred_element_type=jnp.float32)
              for v, AT in zip(vs, ATs)]
        vs = [u / (jnp.sqrt(jnp.sum(u * u)) + 1e-8) for u in us]

    rer_rows, sc_rows = [], []
    for g in range(G):
        rer_acc = jnp.zeros((1, 128), jnp.float32)
        sc_acc = jnp.zeros((1, 128), jnp.float32)
        li = lax.broadcasted_iota(jnp.int32, (1, 128), 1)
        for n in range(2):
            v, AT = vs[2 * g + n], ATs[2 * g + n]
            w = jnp.dot(v, AT, preferred_element_type=jnp.float32)
            sc_val = jnp.sum(w * v)

            v_colv = _col(v, diag)
            gt = (v > v_colv).astype(jnp.float32)
            tie = ((v == v_colv) & (ci < ri)).astype(jnp.float32)
            r = jnp.sum(gt + tie, axis=1, keepdims=True)
            Q = (r == ci.astype(jnp.float32)).astype(jnp.float32)
            sorted_v = jnp.dot(v, Q, preferred_element_type=jnp.float32)

            h = jnp.maximum(
                jnp.dot(sorted_v, W1_ref[...],
                        preferred_element_type=jnp.float32) + b1_ref[...], 0.0)
            o = jnp.sum(h * w2t_ref[...]) + b2_ref[0, 0]
            rer_acc = rer_acc + jnp.where(li == n,
                                          1.0 / (1.0 + jnp.exp(-o)), 0.0)
            sc_acc = sc_acc + jnp.where(li == n, sc_val, 0.0)
        rer_rows.append(rer_acc)
        sc_rows.append(sc_acc)

    rer_ref[0] = jnp.concatenate(rer_rows, axis=0)
    sc_ref[0] = jnp.concatenate(sc_rows, axis=0)


def _run_sgv(normP, cent_u, anc_indices, pos_indices, neg_indices,
             W1, b1, W2, b2):
    in_specs, p_args, c_args = [], [], []
    sel_of = {0: (lambda a, p, n: a), 1: (lambda a, p, n: p),
              2: (lambda a, p, n: n)}
    for g in range(G):
        for s in range(3):
            in_specs.append(pl.BlockSpec(
                (1, K, C),
                lambda t, a, p, n, g=g, s=s: (sel_of[s](a, p, n)[t * G + g],
                                              0, 0)))
            p_args.append(normP)
    for g in range(G):
        for s in range(3):
            in_specs.append(pl.BlockSpec(
                (1, 8, K),
                lambda t, a, p, n, g=g, s=s: (sel_of[s](a, p, n)[t * G + g],
                                              0, 0)))
            c_args.append(cent_u)
    in_specs += [
        pl.BlockSpec((K, K), lambda t, a, p, n: (0, 0)),
        pl.BlockSpec((1, K), lambda t, a, p, n: (0, 0)),
        pl.BlockSpec((1, K), lambda t, a, p, n: (0, 0)),
        pl.BlockSpec(memory_space=pltpu.SMEM),
    ]
    grid_spec = pltpu.PrefetchScalarGridSpec(
        num_scalar_prefetch=3,
        grid=(T // G,),
        in_specs=in_specs,
        out_specs=[
            pl.BlockSpec((1, G, 128), lambda t, a, p, n: (t, 0, 0)),
            pl.BlockSpec((1, G, 128), lambda t, a, p, n: (t, 0, 0)),
        ],
    )
    rer_pad, sc_pad = pl.pallas_call(
        _sgv_body,
        grid_spec=grid_spec,
        out_shape=[
            jax.ShapeDtypeStruct((T // G, G, 128), jnp.float32),
            jax.ShapeDtypeStruct((T // G, G, 128), jnp.float32),
        ],
    )(anc_indices, pos_indices, neg_indices,
      *p_args, *c_args,
      W1, b1.reshape(1, K), W2.reshape(1, K), b2.reshape(1, 1))
    return (rer_pad.reshape(T, 128)[:, :2],
            sc_pad.reshape(T, 128)[:, :2])


def kernel(rt_cls_attn, rt, rt_centroids, shift_and_scale,
           anc_indices, pos_indices, neg_indices,
           W_in, b_in, W1, b1, W2, b2):
    scores = rt_cls_attn[..., 0]
    _, topk_idx = lax.top_k(scores, K)
    topk_idx = jnp.sort(topk_idx, axis=-1)
    rt_k = jnp.take_along_axis(rt, topk_idx[..., None], axis=1)
    cent_k = jnp.take_along_axis(rt_centroids, topk_idx[..., None], axis=1)
    cent_cm = jnp.transpose(cent_k, (0, 2, 1))
    cent_cm = jnp.pad(cent_cm, ((0, 0), (0, 5), (0, 0)))

    normP, cent_u = _run_proj(rt_k, cent_cm, shift_and_scale, W_in, b_in)
    return _run_sgv(normP, cent_u, anc_indices, pos_indices, neg_indices,
                    W1, b1, W2, b2)


# speedup vs baseline: 3.0030x; 1.5373x over previous
import functools

import jax
import jax.numpy as jnp
from jax import lax
from jax.experimental import pallas as pl
from jax.experimental.pallas import tpu as pltpu
from jax.experimental.pallas import tpu_sc as plsc

B = 64
N = 8192
C = 128
K = 256
T = 64
D_THRESH = 0.5
POWER_ITERS = 10


def _splat(x):
    return jnp.full((16,), x, jnp.int32)


def _sc_body(scores_hbm, cent_hbm, rt_hbm, rtk_out, cent_out,
             scores_v, cent_v, hist2d, hist, sfx, pool, gidx_v, rows_v,
             cm_v, sem):
    wid = lax.axis_index("s") * 2 + lax.axis_index("c")
    _LANE = lax.broadcasted_iota(jnp.int32, (16,), 0)

    zf = jnp.zeros((16,), jnp.float32)
    for c in range(3, 8):
        for i in range(16):
            cm_v[c, pl.ds(i * 16, 16)] = zf

    for r in range(2):
        b = r * 32 + wid

        pltpu.sync_copy(scores_hbm.at[b], scores_v)
        pltpu.sync_copy(cent_hbm.at[b], cent_v)

        zi = jnp.zeros((16,), jnp.int32)

        def zero_body(i, _):
            hist2d[pl.ds(i * 16, 16)] = zi
            return 0
        lax.fori_loop(0, 512, zero_body, 0)

        ones = jnp.full((16,), 1, jnp.int32)

        def hist_body(i, _):
            v = scores_v[pl.ds(i * 16, 16)]
            pos = _LANE * 512 + (v * 512.0).astype(jnp.int32)
            plsc.addupdate_scatter(hist2d, [pos], ones)
            return 0
        lax.fori_loop(0, 512, hist_body, 0)

        def red_body(j, _):
            acc = jnp.zeros((16,), jnp.int32)
            for l in range(16):
                acc = acc + hist2d[pl.ds(l * 512 + j * 16, 16)]
            hist[pl.ds(j * 16, 16)] = acc
            return 0
        lax.fori_loop(0, 32, red_body, 0)

        def sfx_body(jj, carry):
            j = 31 - jj
            h = hist[pl.ds(j * 16, 16)]
            rs = lax.rev(plsc.cumsum(lax.rev(h, (0,))), (0,))
            sfx[pl.ds(j * 16, 16)] = rs + _splat(carry)
            return carry + lax.reduce_sum_p.bind(h, axes=(0,))
        lax.fori_loop(0, 32, sfx_body, jnp.int32(0))

        def q_body(j, carry):
            nge, above = carry
            s = sfx[pl.ds(j * 16, 16)]
            nge = nge + lax.reduce_sum_p.bind(
                jnp.where(s >= K, 1, 0), axes=(0,))
            return nge, above
        nge, _ = lax.fori_loop(0, 32, q_body,
                               (jnp.int32(0), jnp.int32(0)))
        q = nge - 1

        def above_body(j, carry):
            idxv = j * 16 + _LANE
            s = sfx[pl.ds(j * 16, 16)]
            return carry + lax.reduce_sum_p.bind(
                jnp.where(idxv == q + 1, s, 0), axes=(0,))
        count_above = lax.fori_loop(0, 32, above_body, jnp.int32(0))

        def pool_body(i, n_pool):
            v = scores_v[pl.ds(i * 16, 16)]
            inq = (v * 512.0).astype(jnp.int32) == q
            incl = plsc.cumsum(jnp.where(inq, 1, 0))
            pos = _splat(n_pool) + incl - 1
            plsc.store_scatter(pool, [pos], v, mask=inq)
            return n_pool + lax.reduce_sum_p.bind(
                jnp.where(inq, 1, 0), axes=(0,))
        n_pool = lax.fori_loop(0, 512, pool_body, jnp.int32(0))
        n_chunks = (n_pool + 15) >> 4

        def bracket_stats(lo, hi):
            mid = (lo + hi) * 0.5

            def body(j, carry):
                mn, mx, cnt = carry
                pb = pool[pl.ds(j * 16, 16)]
                valid = (j * 16 + _LANE) < n_pool
                inb = valid & (pb >= lo) & (pb < hi)
                mn = jnp.minimum(mn, lax.reduce_min_p.bind(
                    jnp.where(inb, pb, jnp.float32(jnp.inf)), axes=(0,)))
                mx = jnp.maximum(mx, lax.reduce_max_p.bind(
                    jnp.where(inb, pb, jnp.float32(-jnp.inf)), axes=(0,)))
                cnt = cnt + lax.reduce_sum_p.bind(
                    jnp.where(valid & (pb >= mid), 1, 0), axes=(0,))
                return mn, mx, cnt
            return mid, lax.fori_loop(
                0, n_chunks, body,
                (jnp.float32(jnp.inf), jnp.float32(-jnp.inf), jnp.int32(0)))

        def bs_cond(lohi):
            return lohi[0] < lohi[1]

        def bs_body(lohi):
            lo, hi = lohi
            mid, (mn, mx, cnt) = bracket_stats(lo, hi)
            done = mn >= mx
            ge = (count_above + cnt) >= K
            new_lo = jnp.where(done, mn, jnp.where(ge, mid,
                                                   jnp.maximum(lo, mn)))
            new_hi = jnp.where(done, mn, jnp.where(ge, hi, mid))
            return new_lo, new_hi
        q_f = q.astype(jnp.float32)
        vstar, _ = lax.while_loop(
            bs_cond, bs_body,
            (q_f * (1.0 / 512.0), (q_f + 1.0) * (1.0 / 512.0)))

        def gtc_body(j, c):
            pb = pool[pl.ds(j * 16, 16)]
            valid = (j * 16 + _LANE) < n_pool
            return c + lax.reduce_sum_p.bind(
                jnp.where(valid & (pb > vstar), 1, 0), axes=(0,))
        cnt_gt = count_above + lax.fori_loop(0, n_chunks, gtc_body,
                                             jnp.int32(0))
        m_eq = K - cnt_gt

        base_splat = _splat(b * N)

        def sel_body(i, carry):
            n_sel, n_eq = carry
            v = scores_v[pl.ds(i * 16, 16)]
            gt = v > vstar
            eq = v == vstar
            incl_eq = plsc.cumsum(jnp.where(eq, 1, 0))
            take_eq = eq & ((incl_eq + _splat(n_eq)) <= m_eq)
            selm = gt | take_eq
            incl_sel = plsc.cumsum(jnp.where(selm, 1, 0))
            pos = _splat(n_sel) + incl_sel - 1
            plsc.store_scatter(gidx_v, [pos],
                               base_splat + i * 16 + _LANE, mask=selm)
            n_sel = n_sel + lax.reduce_sum_p.bind(
                jnp.where(selm, 1, 0), axes=(0,))
            n_eq = n_eq + lax.reduce_sum_p.bind(
                jnp.where(eq, 1, 0), axes=(0,))
            return n_sel, n_eq
        lax.fori_loop(0, 512, sel_body, (jnp.int32(0), jnp.int32(0)))

        for j in range(2):
            pltpu.async_copy(rt_hbm.at[gidx_v.at[pl.ds(j * 128, 128)]],
                             rows_v.at[pl.ds(j * 128, 128)], sem).wait()
        pltpu.sync_copy(rows_v, rtk_out.at[b])

        loc_base = _splat(b * N)
        for i in range(16):
            gi = gidx_v[pl.ds(i * 16, 16)] - loc_base
            for c in range(3):
                vals = plsc.load_gather(cent_v, [gi * 3 + c])
                cm_v[c, pl.ds(i * 16, 16)] = vals
        pltpu.sync_copy(cm_v, cent_out.at[b])


def _run_select_gather(scores, cent_flat, rt_flat):
    kfn = pl.kernel(
        _sc_body,
        out_type=[
            jax.ShapeDtypeStruct((B, K, C), jnp.float32),
            jax.ShapeDtypeStruct((B, 8, K), jnp.float32),
        ],
        mesh=plsc.VectorSubcoreMesh(core_axis_name="c", subcore_axis_name="s"),
        compiler_params=pltpu.CompilerParams(needs_layout_passes=False),
        scratch_types=[
            pltpu.VMEM((N,), jnp.float32),
            pltpu.VMEM((3 * N,), jnp.float32),
            pltpu.VMEM((N,), jnp.int32),
            pltpu.VMEM((512,), jnp.int32),
            pltpu.VMEM((512,), jnp.int32),
            pltpu.VMEM((N,), jnp.float32),
            pltpu.VMEM((K,), jnp.int32),
            pltpu.VMEM((K, C), jnp.float32),
            pltpu.VMEM((8, K), jnp.float32),
            pltpu.SemaphoreType.DMA,
        ],
    )
    return kfn(scores, cent_flat, rt_flat)


def _proj_body(rtk_ref, W_ref, b_ref, cent_ref, ss_ref, normP_ref, centu_ref):
    b = pl.program_id(0)
    rtk = rtk_ref[0]
    proj = jnp.dot(rtk, W_ref[...], preferred_element_type=jnp.float32)
    proj = proj + b_ref[...]
    nrm = jnp.sqrt(jnp.sum(proj * proj, axis=1, keepdims=True))
    normP_ref[0] = proj / (nrm + 1e-6)

    cent = cent_ref[0]
    scale = ss_ref[b, 3]
    ri = lax.broadcasted_iota(jnp.int32, (8, K), 0)
    shift = jnp.where(ri == 0, ss_ref[b, 0],
            jnp.where(ri == 1, ss_ref[b, 1],
            jnp.where(ri == 2, ss_ref[b, 2], 0.0)))
    centu_ref[0] = cent * scale + shift


def _run_proj(rt_k, cent_cm, shift_and_scale, W_in, b_in):
    return pl.pallas_call(
        _proj_body,
        grid=(B,),
        in_specs=[
            pl.BlockSpec((1, K, C), lambda b: (b, 0, 0)),
            pl.BlockSpec((C, C), lambda b: (0, 0)),
            pl.BlockSpec((1, C), lambda b: (0, 0)),
            pl.BlockSpec((1, 8, K), lambda b: (b, 0, 0)),
            pl.BlockSpec(memory_space=pltpu.SMEM),
        ],
        out_specs=[
            pl.BlockSpec((1, K, C), lambda b: (b, 0, 0)),
            pl.BlockSpec((1, 8, K), lambda b: (b, 0, 0)),
        ],
        out_shape=[
            jax.ShapeDtypeStruct((B, K, C), jnp.float32),
            jax.ShapeDtypeStruct((B, 8, K), jnp.float32),
        ],
    )(rt_k, W_in, b_in.reshape(1, C), cent_cm, shift_and_scale)


def _col(row, diag):
    return jnp.sum(jnp.where(diag, row, 0.0), axis=1, keepdims=True)


def _pdist(cm, diag):
    acc = jnp.zeros((K, K), jnp.float32)
    for c in range(3):
        xj = cm[c:c + 1, :]
        xi = _col(xj, diag)
        d = xi - xj
        acc = acc + d * d
    return jnp.sqrt(acc + 1e-12)


G = 4


def _sgv_body(anc_i, pos_i, neg_i, *refs):
    del anc_i, pos_i, neg_i
    P_refs = refs[0:3 * G]
    C_refs = refs[3 * G:6 * G]
    W1_ref, b1_ref, w2t_ref, b2_ref = refs[6 * G:6 * G + 4]
    rer_ref, sc_ref = refs[6 * G + 4:]

    ri = lax.broadcasted_iota(jnp.int32, (K, K), 0)
    ci = lax.broadcasted_iota(jnp.int32, (K, K), 1)
    diag = ri == ci

    ATs = []
    for g in range(G):
        anc_n = P_refs[3 * g][0]
        d_anc = _pdist(C_refs[3 * g][0], diag)
        for n in (1, 2):
            nn_n = P_refs[3 * g + n][0]
            d_nn = _pdist(C_refs[3 * g + n][0], diag)
            beta = (d_anc - d_nn) ** 2 * (1.0 / (D_THRESH * D_THRESH))
            geo = jnp.maximum(1.0 - beta, 0.0)
            featT = lax.dot_general(nn_n, anc_n, (((1,), (1,)), ((), ())),
                                    preferred_element_type=jnp.float32)
            ATs.append(jnp.where(diag, 0.0, geo * jnp.maximum(featT, 0.0)))

    vs = [jnp.full((1, K), 1.0 / (K ** 0.5), jnp.float32)] * (2 * G)
    for _ in range(POWER_ITERS):
        us = [jnp.dot(v, AT, preferred_element_type=jnp.float32)
              for v, AT in zip(vs, ATs)]
        vs = [u / (jnp.sqrt(jnp.sum(u * u)) + 1e-8) for u in us]

    rer_rows, sc_rows = [], []
    for g in range(G):
        rer_acc = jnp.zeros((1, 128), jnp.float32)
        sc_acc = jnp.zeros((1, 128), jnp.float32)
        li = lax.broadcasted_iota(jnp.int32, (1, 128), 1)
        for n in range(2):
            v, AT = vs[2 * g + n], ATs[2 * g + n]
            w = jnp.dot(v, AT, preferred_element_type=jnp.float32)
            sc_val = jnp.sum(w * v)

            v_colv = _col(v, diag)
            gt = (v > v_colv).astype(jnp.float32)
            tie = ((v == v_colv) & (ci < ri)).astype(jnp.float32)
            r = jnp.sum(gt + tie, axis=1, keepdims=True)
            Q = (r == ci.astype(jnp.float32)).astype(jnp.float32)
            sorted_v = jnp.dot(v, Q, preferred_element_type=jnp.float32)

            h = jnp.maximum(
                jnp.dot(sorted_v, W1_ref[...],
                        preferred_element_type=jnp.float32) + b1_ref[...], 0.0)
            o = jnp.sum(h * w2t_ref[...]) + b2_ref[0, 0]
            rer_acc = rer_acc + jnp.where(li == n,
                                          1.0 / (1.0 + jnp.exp(-o)), 0.0)
            sc_acc = sc_acc + jnp.where(li == n, sc_val, 0.0)
        rer_rows.append(rer_acc)
        sc_rows.append(sc_acc)

    rer_ref[0] = jnp.concatenate(rer_rows, axis=0)
    sc_ref[0] = jnp.concatenate(sc_rows, axis=0)


def _run_sgv(normP, cent_u, anc_indices, pos_indices, neg_indices,
             W1, b1, W2, b2):
    in_specs, p_args, c_args = [], [], []
    sel_of = {0: (lambda a, p, n: a), 1: (lambda a, p, n: p),
              2: (lambda a, p, n: n)}
    for g in range(G):
        for s in range(3):
            in_specs.append(pl.BlockSpec(
                (1, K, C),
                lambda t, a, p, n, g=g, s=s: (sel_of[s](a, p, n)[t * G + g],
                                              0, 0)))
            p_args.append(normP)
    for g in range(G):
        for s in range(3):
            in_specs.append(pl.BlockSpec(
                (1, 8, K),
                lambda t, a, p, n, g=g, s=s: (sel_of[s](a, p, n)[t * G + g],
                                              0, 0)))
            c_args.append(cent_u)
    in_specs += [
        pl.BlockSpec((K, K), lambda t, a, p, n: (0, 0)),
        pl.BlockSpec((1, K), lambda t, a, p, n: (0, 0)),
        pl.BlockSpec((1, K), lambda t, a, p, n: (0, 0)),
        pl.BlockSpec(memory_space=pltpu.SMEM),
    ]
    grid_spec = pltpu.PrefetchScalarGridSpec(
        num_scalar_prefetch=3,
        grid=(T // G,),
        in_specs=in_specs,
        out_specs=[
            pl.BlockSpec((1, G, 128), lambda t, a, p, n: (t, 0, 0)),
            pl.BlockSpec((1, G, 128), lambda t, a, p, n: (t, 0, 0)),
        ],
    )
    rer_pad, sc_pad = pl.pallas_call(
        _sgv_body,
        grid_spec=grid_spec,
        out_shape=[
            jax.ShapeDtypeStruct((T // G, G, 128), jnp.float32),
            jax.ShapeDtypeStruct((T // G, G, 128), jnp.float32),
        ],
    )(anc_indices, pos_indices, neg_indices,
      *p_args, *c_args,
      W1, b1.reshape(1, K), W2.reshape(1, K), b2.reshape(1, 1))
    return (rer_pad.reshape(T, 128)[:, :2],
            sc_pad.reshape(T, 128)[:, :2])


def kernel(rt_cls_attn, rt, rt_centroids, shift_and_scale,
           anc_indices, pos_indices, neg_indices,
           W_in, b_in, W1, b1, W2, b2):
    scores = rt_cls_attn[..., 0]
    rt_k, cent_cm = _run_select_gather(
        scores, rt_centroids.reshape(B, 3 * N), rt.reshape(B * N, C))

    normP, cent_u = _run_proj(rt_k, cent_cm, shift_and_scale, W_in, b_in)
    return _run_sgv(normP, cent_u, anc_indices, pos_indices, neg_indices,
                    W1, b1, W2, b2)


# speedup vs baseline: 3.0724x; 1.0231x over previous
import functools

import jax
import jax.numpy as jnp
from jax import lax
from jax.experimental import pallas as pl
from jax.experimental.pallas import tpu as pltpu
from jax.experimental.pallas import tpu_sc as plsc

B = 64
N = 8192
C = 128
K = 256
T = 64
D_THRESH = 0.5
POWER_ITERS = 10


def _splat(x):
    return jnp.full((16,), x, jnp.int32)


def _sc_body(scores_hbm, cent_hbm, rt_hbm, rtk_out, cent_out,
             scores_v, cent_v, hist2d, hist, sfx, pool, gidx_v, rows_v,
             cm_v, sem):
    wid = lax.axis_index("s") * 2 + lax.axis_index("c")
    _LANE = lax.broadcasted_iota(jnp.int32, (16,), 0)

    zf = jnp.zeros((16,), jnp.float32)
    for c in range(3, 8):
        for i in range(16):
            cm_v[c, pl.ds(i * 16, 16)] = zf

    for r in range(2):
        b = r * 32 + wid

        pltpu.sync_copy(scores_hbm.at[b], scores_v)
        pltpu.sync_copy(cent_hbm.at[b], cent_v)

        zi = jnp.zeros((16,), jnp.int32)

        def zero_body(i, _):
            hist2d[pl.ds(i * 16, 16)] = zi
            return 0
        lax.fori_loop(0, 512, zero_body, 0)

        ones = jnp.full((16,), 1, jnp.int32)

        def hist_body(i, _):
            v = scores_v[pl.ds(i * 16, 16)]
            pos = _LANE * 512 + (v * 512.0).astype(jnp.int32)
            plsc.addupdate_scatter(hist2d, [pos], ones)
            return 0
        lax.fori_loop(0, 512, hist_body, 0)

        def red_body(j, _):
            acc = jnp.zeros((16,), jnp.int32)
            for l in range(16):
                acc = acc + hist2d[pl.ds(l * 512 + j * 16, 16)]
            hist[pl.ds(j * 16, 16)] = acc
            return 0
        lax.fori_loop(0, 32, red_body, 0)

        def sfx_body(jj, carry):
            j = 31 - jj
            h = hist[pl.ds(j * 16, 16)]
            rs = lax.rev(plsc.cumsum(lax.rev(h, (0,))), (0,))
            sfx[pl.ds(j * 16, 16)] = rs + _splat(carry)
            return carry + lax.reduce_sum_p.bind(h, axes=(0,))
        lax.fori_loop(0, 32, sfx_body, jnp.int32(0))

        def q_body(j, carry):
            nge, above = carry
            s = sfx[pl.ds(j * 16, 16)]
            nge = nge + lax.reduce_sum_p.bind(
                jnp.where(s >= K, 1, 0), axes=(0,))
            return nge, above
        nge, _ = lax.fori_loop(0, 32, q_body,
                               (jnp.int32(0), jnp.int32(0)))
        q = nge - 1

        def above_body(j, carry):
            idxv = j * 16 + _LANE
            s = sfx[pl.ds(j * 16, 16)]
            return carry + lax.reduce_sum_p.bind(
                jnp.where(idxv == q + 1, s, 0), axes=(0,))
        count_above = lax.fori_loop(0, 32, above_body, jnp.int32(0))

        def pool_body(i, n_pool):
            v = scores_v[pl.ds(i * 16, 16)]
            inq = (v * 512.0).astype(jnp.int32) == q
            incl = plsc.cumsum(jnp.where(inq, 1, 0))
            pos = _splat(n_pool) + incl - 1
            plsc.store_scatter(pool, [pos], v, mask=inq)
            return n_pool + lax.reduce_sum_p.bind(
                jnp.where(inq, 1, 0), axes=(0,))
        n_pool = lax.fori_loop(0, 512, pool_body, jnp.int32(0))
        n_chunks = (n_pool + 15) >> 4

        def bracket_stats(lo, hi):
            mid = (lo + hi) * 0.5

            def body(j, carry):
                mn, mx, cnt = carry
                pb = pool[pl.ds(j * 16, 16)]
                valid = (j * 16 + _LANE) < n_pool
                inb = valid & (pb >= lo) & (pb < hi)
                mn = jnp.minimum(mn, lax.reduce_min_p.bind(
                    jnp.where(inb, pb, jnp.float32(jnp.inf)), axes=(0,)))
                mx = jnp.maximum(mx, lax.reduce_max_p.bind(
                    jnp.where(inb, pb, jnp.float32(-jnp.inf)), axes=(0,)))
                cnt = cnt + lax.reduce_sum_p.bind(
                    jnp.where(valid & (pb >= mid), 1, 0), axes=(0,))
                return mn, mx, cnt
            return mid, lax.fori_loop(
                0, n_chunks, body,
                (jnp.float32(jnp.inf), jnp.float32(-jnp.inf), jnp.int32(0)))

        def bs_cond(lohi):
            return lohi[0] < lohi[1]

        def bs_body(lohi):
            lo, hi = lohi
            mid, (mn, mx, cnt) = bracket_stats(lo, hi)
            done = mn >= mx
            ge = (count_above + cnt) >= K
            new_lo = jnp.where(done, mn, jnp.where(ge, mid,
                                                   jnp.maximum(lo, mn)))
            new_hi = jnp.where(done, mn, jnp.where(ge, hi, mid))
            return new_lo, new_hi
        q_f = q.astype(jnp.float32)
        vstar, _ = lax.while_loop(
            bs_cond, bs_body,
            (q_f * (1.0 / 512.0), (q_f + 1.0) * (1.0 / 512.0)))

        def gtc_body(j, c):
            pb = pool[pl.ds(j * 16, 16)]
            valid = (j * 16 + _LANE) < n_pool
            return c + lax.reduce_sum_p.bind(
                jnp.where(valid & (pb > vstar), 1, 0), axes=(0,))
        cnt_gt = count_above + lax.fori_loop(0, n_chunks, gtc_body,
                                             jnp.int32(0))
        m_eq = K - cnt_gt

        base_splat = _splat(b * N)

        def sel_body(i, carry):
            n_sel, n_eq = carry
            v = scores_v[pl.ds(i * 16, 16)]
            gt = v > vstar
            eq = v == vstar
            incl_eq = plsc.cumsum(jnp.where(eq, 1, 0))
            take_eq = eq & ((incl_eq + _splat(n_eq)) <= m_eq)
            selm = gt | take_eq
            incl_sel = plsc.cumsum(jnp.where(selm, 1, 0))
            pos = _splat(n_sel) + incl_sel - 1
            plsc.store_scatter(gidx_v, [pos],
                               base_splat + i * 16 + _LANE, mask=selm)
            n_sel = n_sel + lax.reduce_sum_p.bind(
                jnp.where(selm, 1, 0), axes=(0,))
            n_eq = n_eq + lax.reduce_sum_p.bind(
                jnp.where(eq, 1, 0), axes=(0,))
            return n_sel, n_eq
        lax.fori_loop(0, 512, sel_body, (jnp.int32(0), jnp.int32(0)))

        for j in range(2):
            pltpu.async_copy(rt_hbm.at[gidx_v.at[pl.ds(j * 128, 128)]],
                             rows_v.at[pl.ds(j * 128, 128)], sem).wait()
        pltpu.sync_copy(rows_v, rtk_out.at[b])

        loc_base = _splat(b * N)
        for i in range(16):
            gi = gidx_v[pl.ds(i * 16, 16)] - loc_base
            for c in range(3):
                vals = plsc.load_gather(cent_v, [gi * 3 + c])
                cm_v[c, pl.ds(i * 16, 16)] = vals
        pltpu.sync_copy(cm_v, cent_out.at[b])


def _run_select_gather(scores, cent_flat, rt_flat):
    kfn = pl.kernel(
        _sc_body,
        out_type=[
            jax.ShapeDtypeStruct((B, K, C), jnp.float32),
            jax.ShapeDtypeStruct((B, 8, K), jnp.float32),
        ],
        mesh=plsc.VectorSubcoreMesh(core_axis_name="c", subcore_axis_name="s"),
        compiler_params=pltpu.CompilerParams(needs_layout_passes=False),
        scratch_types=[
            pltpu.VMEM((N,), jnp.float32),
            pltpu.VMEM((3 * N,), jnp.float32),
            pltpu.VMEM((N,), jnp.int32),
            pltpu.VMEM((512,), jnp.int32),
            pltpu.VMEM((512,), jnp.int32),
            pltpu.VMEM((N,), jnp.float32),
            pltpu.VMEM((K,), jnp.int32),
            pltpu.VMEM((K, C), jnp.float32),
            pltpu.VMEM((8, K), jnp.float32),
            pltpu.SemaphoreType.DMA,
        ],
    )
    return kfn(scores, cent_flat, rt_flat)


def _proj_body(rtk_ref, W_ref, b_ref, cent_ref, ss_ref, normP_ref, centu_ref):
    b = pl.program_id(0)
    rtk = rtk_ref[0]
    proj = jnp.dot(rtk, W_ref[...], preferred_element_type=jnp.float32)
    proj = proj + b_ref[...]
    nrm = jnp.sqrt(jnp.sum(proj * proj, axis=1, keepdims=True))
    normP_ref[0] = proj / (nrm + 1e-6)

    cent = cent_ref[0]
    scale = ss_ref[b, 3]
    ri = lax.broadcasted_iota(jnp.int32, (8, K), 0)
    shift = jnp.where(ri == 0, ss_ref[b, 0],
            jnp.where(ri == 1, ss_ref[b, 1],
            jnp.where(ri == 2, ss_ref[b, 2], 0.0)))
    centu_ref[0] = cent * scale + shift


def _run_proj(rt_k, cent_cm, shift_and_scale, W_in, b_in):
    return pl.pallas_call(
        _proj_body,
        grid=(B,),
        in_specs=[
            pl.BlockSpec((1, K, C), lambda b: (b, 0, 0)),
            pl.BlockSpec((C, C), lambda b: (0, 0)),
            pl.BlockSpec((1, C), lambda b: (0, 0)),
            pl.BlockSpec((1, 8, K), lambda b: (b, 0, 0)),
            pl.BlockSpec(memory_space=pltpu.SMEM),
        ],
        out_specs=[
            pl.BlockSpec((1, K, C), lambda b: (b, 0, 0)),
            pl.BlockSpec((1, 8, K), lambda b: (b, 0, 0)),
        ],
        out_shape=[
            jax.ShapeDtypeStruct((B, K, C), jnp.float32),
            jax.ShapeDtypeStruct((B, 8, K), jnp.float32),
        ],
    )(rt_k, W_in, b_in.reshape(1, C), cent_cm, shift_and_scale)


def _col(row, diag):
    return jnp.sum(jnp.where(diag, row, 0.0), axis=1, keepdims=True)


def _pdist(cm, diag):
    acc = jnp.zeros((K, K), jnp.float32)
    for c in range(3):
        xj = cm[c:c + 1, :]
        xi = _col(xj, diag)
        d = xi - xj
        acc = acc + d * d
    return jnp.sqrt(acc + 1e-12)


G = 8


def _sgv_body(anc_i, pos_i, neg_i, *refs):
    del anc_i, pos_i, neg_i
    P_refs = refs[0:3 * G]
    C_refs = refs[3 * G:6 * G]
    W1_ref, b1_ref, w2t_ref, b2_ref = refs[6 * G:6 * G + 4]
    rer_ref, sc_ref = refs[6 * G + 4:]

    ri = lax.broadcasted_iota(jnp.int32, (K, K), 0)
    ci = lax.broadcasted_iota(jnp.int32, (K, K), 1)
    diag = ri == ci

    ATs = []
    for g in range(G):
        anc_n = P_refs[3 * g][0]
        d_anc = _pdist(C_refs[3 * g][0], diag)
        for n in (1, 2):
            nn_n = P_refs[3 * g + n][0]
            d_nn = _pdist(C_refs[3 * g + n][0], diag)
            beta = (d_anc - d_nn) ** 2 * (1.0 / (D_THRESH * D_THRESH))
            geo = jnp.maximum(1.0 - beta, 0.0)
            featT = lax.dot_general(nn_n, anc_n, (((1,), (1,)), ((), ())),
                                    preferred_element_type=jnp.float32)
            ATs.append(jnp.where(diag, 0.0, geo * jnp.maximum(featT, 0.0)))

    vs = [jnp.full((1, K), 1.0 / (K ** 0.5), jnp.float32)] * (2 * G)
    for _ in range(POWER_ITERS):
        us = [jnp.dot(v, AT, preferred_element_type=jnp.float32)
              for v, AT in zip(vs, ATs)]
        vs = [u / (jnp.sqrt(jnp.sum(u * u)) + 1e-8) for u in us]

    rer_rows, sc_rows = [], []
    for g in range(G):
        rer_acc = jnp.zeros((1, 128), jnp.float32)
        sc_acc = jnp.zeros((1, 128), jnp.float32)
        li = lax.broadcasted_iota(jnp.int32, (1, 128), 1)
        for n in range(2):
            v, AT = vs[2 * g + n], ATs[2 * g + n]
            w = jnp.dot(v, AT, preferred_element_type=jnp.float32)
            sc_val = jnp.sum(w * v)

            v_colv = _col(v, diag)
            gt = (v > v_colv).astype(jnp.float32)
            tie = ((v == v_colv) & (ci < ri)).astype(jnp.float32)
            r = jnp.sum(gt + tie, axis=1, keepdims=True)
            Q = (r == ci.astype(jnp.float32)).astype(jnp.float32)
            sorted_v = jnp.dot(v, Q, preferred_element_type=jnp.float32)

            h = jnp.maximum(
                jnp.dot(sorted_v, W1_ref[...],
                        preferred_element_type=jnp.float32) + b1_ref[...], 0.0)
            o = jnp.sum(h * w2t_ref[...]) + b2_ref[0, 0]
            rer_acc = rer_acc + jnp.where(li == n,
                                          1.0 / (1.0 + jnp.exp(-o)), 0.0)
            sc_acc = sc_acc + jnp.where(li == n, sc_val, 0.0)
        rer_rows.append(rer_acc)
        sc_rows.append(sc_acc)

    rer_ref[0] = jnp.concatenate(rer_rows, axis=0)
    sc_ref[0] = jnp.concatenate(sc_rows, axis=0)


def _run_sgv(normP, cent_u, anc_indices, pos_indices, neg_indices,
             W1, b1, W2, b2):
    in_specs, p_args, c_args = [], [], []
    sel_of = {0: (lambda a, p, n: a), 1: (lambda a, p, n: p),
              2: (lambda a, p, n: n)}
    for g in range(G):
        for s in range(3):
            in_specs.append(pl.BlockSpec(
                (1, K, C),
                lambda t, a, p, n, g=g, s=s: (sel_of[s](a, p, n)[t * G + g],
                                              0, 0)))
            p_args.append(normP)
    for g in range(G):
        for s in range(3):
            in_specs.append(pl.BlockSpec(
                (1, 8, K),
                lambda t, a, p, n, g=g, s=s: (sel_of[s](a, p, n)[t * G + g],
                                              0, 0)))
            c_args.append(cent_u)
    in_specs += [
        pl.BlockSpec((K, K), lambda t, a, p, n: (0, 0)),
        pl.BlockSpec((1, K), lambda t, a, p, n: (0, 0)),
        pl.BlockSpec((1, K), lambda t, a, p, n: (0, 0)),
        pl.BlockSpec(memory_space=pltpu.SMEM),
    ]
    grid_spec = pltpu.PrefetchScalarGridSpec(
        num_scalar_prefetch=3,
        grid=(T // G,),
        in_specs=in_specs,
        out_specs=[
            pl.BlockSpec((1, G, 128), lambda t, a, p, n: (t, 0, 0)),
            pl.BlockSpec((1, G, 128), lambda t, a, p, n: (t, 0, 0)),
        ],
    )
    rer_pad, sc_pad = pl.pallas_call(
        _sgv_body,
        grid_spec=grid_spec,
        out_shape=[
            jax.ShapeDtypeStruct((T // G, G, 128), jnp.float32),
            jax.ShapeDtypeStruct((T // G, G, 128), jnp.float32),
        ],
    )(anc_indices, pos_indices, neg_indices,
      *p_args, *c_args,
      W1, b1.reshape(1, K), W2.reshape(1, K), b2.reshape(1, 1))
    return (rer_pad.reshape(T, 128)[:, :2],
            sc_pad.reshape(T, 128)[:, :2])


def kernel(rt_cls_attn, rt, rt_centroids, shift_and_scale,
           anc_indices, pos_indices, neg_indices,
           W_in, b_in, W1, b1, W2, b2):
    scores = rt_cls_attn[..., 0]
    rt_k, cent_cm = _run_select_gather(
        scores, rt_centroids.reshape(B, 3 * N), rt.reshape(B * N, C))

    normP, cent_u = _run_proj(rt_k, cent_cm, shift_and_scale, W_in, b_in)
    return _run_sgv(normP, cent_u, anc_indices, pos_indices, neg_indices,
                    W1, b1, W2, b2)


# speedup vs baseline: 3.1013x; 1.0094x over previous
import functools

import jax
import jax.numpy as jnp
from jax import lax
from jax.experimental import pallas as pl
from jax.experimental.pallas import tpu as pltpu
from jax.experimental.pallas import tpu_sc as plsc

B = 64
N = 8192
C = 128
K = 256
T = 64
D_THRESH = 0.5
POWER_ITERS = 10


def _splat(x):
    return jnp.full((16,), x, jnp.int32)


def _sc_body(scores_hbm, cent_hbm, rt_hbm, rtk_out, cent_out,
             scores_v, cent_v, hist2d, hist, sfx, pool, gidx_v, rows_v,
             cm_v, sem):
    wid = lax.axis_index("s") * 2 + lax.axis_index("c")
    _LANE = lax.broadcasted_iota(jnp.int32, (16,), 0)

    zf = jnp.zeros((16,), jnp.float32)
    for c in range(3, 8):
        for i in range(16):
            cm_v[c, pl.ds(i * 16, 16)] = zf

    zi = jnp.zeros((16,), jnp.int32)

    def zero_body(i, _):
        hist2d[pl.ds(i * 16, 16)] = zi
        return 0
    lax.fori_loop(0, 512, zero_body, 0)

    for r in range(2):
        b = r * 32 + wid

        pltpu.sync_copy(scores_hbm.at[b], scores_v)
        pltpu.sync_copy(cent_hbm.at[b], cent_v)

        ones = jnp.full((16,), 1, jnp.int32)

        def hist_body(i, _):
            for u in range(4):
                v = scores_v[pl.ds((i * 4 + u) * 16, 16)]
                pos = _LANE * 512 + (v * 512.0).astype(jnp.int32)
                plsc.addupdate_scatter(hist2d, [pos], ones)
            return 0
        lax.fori_loop(0, 128, hist_body, 0)

        def red_body(j, _):
            acc = jnp.zeros((16,), jnp.int32)
            for l in range(16):
                acc = acc + hist2d[pl.ds(l * 512 + j * 16, 16)]
                hist2d[pl.ds(l * 512 + j * 16, 16)] = zi
            hist[pl.ds(j * 16, 16)] = acc
            return 0
        lax.fori_loop(0, 32, red_body, 0)

        def sfx_body(jj, carry):
            j = 31 - jj
            h = hist[pl.ds(j * 16, 16)]
            rs = lax.rev(plsc.cumsum(lax.rev(h, (0,))), (0,))
            sfx[pl.ds(j * 16, 16)] = rs + _splat(carry)
            return carry + lax.reduce_sum_p.bind(h, axes=(0,))
        lax.fori_loop(0, 32, sfx_body, jnp.int32(0))

        def q_body(j, carry):
            nge, above = carry
            s = sfx[pl.ds(j * 16, 16)]
            nge = nge + lax.reduce_sum_p.bind(
                jnp.where(s >= K, 1, 0), axes=(0,))
            return nge, above
        nge, _ = lax.fori_loop(0, 32, q_body,
                               (jnp.int32(0), jnp.int32(0)))
        q = nge - 1

        def above_body(j, carry):
            idxv = j * 16 + _LANE
            s = sfx[pl.ds(j * 16, 16)]
            return carry + lax.reduce_sum_p.bind(
                jnp.where(idxv == q + 1, s, 0), axes=(0,))
        count_above = lax.fori_loop(0, 32, above_body, jnp.int32(0))

        def pool_body(i, n_pool):
            for u in range(4):
                v = scores_v[pl.ds((i * 4 + u) * 16, 16)]
                inq = (v * 512.0).astype(jnp.int32) == q
                incl = plsc.cumsum(jnp.where(inq, 1, 0))
                pos = _splat(n_pool) + incl - 1
                plsc.store_scatter(pool, [pos], v, mask=inq)
                n_pool = n_pool + lax.reduce_sum_p.bind(
                    jnp.where(inq, 1, 0), axes=(0,))
            return n_pool
        n_pool = lax.fori_loop(0, 128, pool_body, jnp.int32(0))
        n_chunks = (n_pool + 15) >> 4

        def bracket_stats(lo, hi):
            mid = (lo + hi) * 0.5

            def body(j, carry):
                mn, mx, cnt = carry
                pb = pool[pl.ds(j * 16, 16)]
                valid = (j * 16 + _LANE) < n_pool
                inb = valid & (pb >= lo) & (pb < hi)
                mn = jnp.minimum(mn, lax.reduce_min_p.bind(
                    jnp.where(inb, pb, jnp.float32(jnp.inf)), axes=(0,)))
                mx = jnp.maximum(mx, lax.reduce_max_p.bind(
                    jnp.where(inb, pb, jnp.float32(-jnp.inf)), axes=(0,)))
                cnt = cnt + lax.reduce_sum_p.bind(
                    jnp.where(valid & (pb >= mid), 1, 0), axes=(0,))
                return mn, mx, cnt
            return mid, lax.fori_loop(
                0, n_chunks, body,
                (jnp.float32(jnp.inf), jnp.float32(-jnp.inf), jnp.int32(0)))

        def bs_cond(lohi):
            return lohi[0] < lohi[1]

        def bs_body(lohi):
            lo, hi = lohi
            mid, (mn, mx, cnt) = bracket_stats(lo, hi)
            done = mn >= mx
            ge = (count_above + cnt) >= K
            new_lo = jnp.where(done, mn, jnp.where(ge, mid,
                                                   jnp.maximum(lo, mn)))
            new_hi = jnp.where(done, mn, jnp.where(ge, hi, mid))
            return new_lo, new_hi
        q_f = q.astype(jnp.float32)
        vstar, _ = lax.while_loop(
            bs_cond, bs_body,
            (q_f * (1.0 / 512.0), (q_f + 1.0) * (1.0 / 512.0)))

        def gtc_body(j, c):
            pb = pool[pl.ds(j * 16, 16)]
            valid = (j * 16 + _LANE) < n_pool
            return c + lax.reduce_sum_p.bind(
                jnp.where(valid & (pb > vstar), 1, 0), axes=(0,))
        cnt_gt = count_above + lax.fori_loop(0, n_chunks, gtc_body,
                                             jnp.int32(0))
        m_eq = K - cnt_gt

        base_splat = _splat(b * N)

        def sel_body(i, carry):
            n_sel, n_eq = carry
            for u in range(4):
                ii = i * 4 + u
                v = scores_v[pl.ds(ii * 16, 16)]
                gt = v > vstar
                eq = v == vstar
                incl_eq = plsc.cumsum(jnp.where(eq, 1, 0))
                take_eq = eq & ((incl_eq + _splat(n_eq)) <= m_eq)
                selm = gt | take_eq
                incl_sel = plsc.cumsum(jnp.where(selm, 1, 0))
                pos = _splat(n_sel) + incl_sel - 1
                plsc.store_scatter(gidx_v, [pos],
                                   base_splat + ii * 16 + _LANE, mask=selm)
                n_sel = n_sel + lax.reduce_sum_p.bind(
                    jnp.where(selm, 1, 0), axes=(0,))
                n_eq = n_eq + lax.reduce_sum_p.bind(
                    jnp.where(eq, 1, 0), axes=(0,))
            return n_sel, n_eq
        lax.fori_loop(0, 128, sel_body, (jnp.int32(0), jnp.int32(0)))

        for j in range(2):
            pltpu.async_copy(rt_hbm.at[gidx_v.at[pl.ds(j * 128, 128)]],
                             rows_v.at[pl.ds(j * 128, 128)], sem).wait()
        pltpu.sync_copy(rows_v, rtk_out.at[b])

        loc_base = _splat(b * N)
        for i in range(16):
            gi = gidx_v[pl.ds(i * 16, 16)] - loc_base
            for c in range(3):
                vals = plsc.load_gather(cent_v, [gi * 3 + c])
                cm_v[c, pl.ds(i * 16, 16)] = vals
        pltpu.sync_copy(cm_v, cent_out.at[b])


def _run_select_gather(scores, cent_flat, rt_flat):
    kfn = pl.kernel(
        _sc_body,
        out_type=[
            jax.ShapeDtypeStruct((B, K, C), jnp.float32),
            jax.ShapeDtypeStruct((B, 8, K), jnp.float32),
        ],
        mesh=plsc.VectorSubcoreMesh(core_axis_name="c", subcore_axis_name="s"),
        compiler_params=pltpu.CompilerParams(needs_layout_passes=False),
        scratch_types=[
            pltpu.VMEM((N,), jnp.float32),
            pltpu.VMEM((3 * N,), jnp.float32),
            pltpu.VMEM((N,), jnp.int32),
            pltpu.VMEM((512,), jnp.int32),
            pltpu.VMEM((512,), jnp.int32),
            pltpu.VMEM((N,), jnp.float32),
            pltpu.VMEM((K,), jnp.int32),
            pltpu.VMEM((K, C), jnp.float32),
            pltpu.VMEM((8, K), jnp.float32),
            pltpu.SemaphoreType.DMA,
        ],
    )
    return kfn(scores, cent_flat, rt_flat)


def _proj_body(rtk_ref, W_ref, b_ref, cent_ref, ss_ref, normP_ref, centu_ref):
    b = pl.program_id(0)
    rtk = rtk_ref[0]
    proj = jnp.dot(rtk, W_ref[...], preferred_element_type=jnp.float32)
    proj = proj + b_ref[...]
    nrm = jnp.sqrt(jnp.sum(proj * proj, axis=1, keepdims=True))
    normP_ref[0] = proj / (nrm + 1e-6)

    cent = cent_ref[0]
    scale = ss_ref[b, 3]
    ri = lax.broadcasted_iota(jnp.int32, (8, K), 0)
    shift = jnp.where(ri == 0, ss_ref[b, 0],
            jnp.where(ri == 1, ss_ref[b, 1],
            jnp.where(ri == 2, ss_ref[b, 2], 0.0)))
    centu_ref[0] = cent * scale + shift


def _run_proj(rt_k, cent_cm, shift_and_scale, W_in, b_in):
    return pl.pallas_call(
        _proj_body,
        grid=(B,),
        in_specs=[
            pl.BlockSpec((1, K, C), lambda b: (b, 0, 0)),
            pl.BlockSpec((C, C), lambda b: (0, 0)),
            pl.BlockSpec((1, C), lambda b: (0, 0)),
            pl.BlockSpec((1, 8, K), lambda b: (b, 0, 0)),
            pl.BlockSpec(memory_space=pltpu.SMEM),
        ],
        out_specs=[
            pl.BlockSpec((1, K, C), lambda b: (b, 0, 0)),
            pl.BlockSpec((1, 8, K), lambda b: (b, 0, 0)),
        ],
        out_shape=[
            jax.ShapeDtypeStruct((B, K, C), jnp.float32),
            jax.ShapeDtypeStruct((B, 8, K), jnp.float32),
        ],
    )(rt_k, W_in, b_in.reshape(1, C), cent_cm, shift_and_scale)


def _col(row, diag):
    return jnp.sum(jnp.where(diag, row, 0.0), axis=1, keepdims=True)


def _pdist(cm, diag):
    acc = jnp.zeros((K, K), jnp.float32)
    for c in range(3):
        xj = cm[c:c + 1, :]
        xi = _col(xj, diag)
        d = xi - xj
        acc = acc + d * d
    return jnp.sqrt(acc + 1e-12)


G = 8


def _sgv_body(anc_i, pos_i, neg_i, *refs):
    del anc_i, pos_i, neg_i
    P_refs = refs[0:3 * G]
    C_refs = refs[3 * G:6 * G]
    W1_ref, b1_ref, w2t_ref, b2_ref = refs[6 * G:6 * G + 4]
    rer_ref, sc_ref = refs[6 * G + 4:]

    ri = lax.broadcasted_iota(jnp.int32, (K, K), 0)
    ci = lax.broadcasted_iota(jnp.int32, (K, K), 1)
    diag = ri == ci

    ATs = []
    for g in range(G):
        anc_n = P_refs[3 * g][0]
        d_anc = _pdist(C_refs[3 * g][0], diag)
        for n in (1, 2):
            nn_n = P_refs[3 * g + n][0]
            d_nn = _pdist(C_refs[3 * g + n][0], diag)
            beta = (d_anc - d_nn) ** 2 * (1.0 / (D_THRESH * D_THRESH))
            geo = jnp.maximum(1.0 - beta, 0.0)
            featT = lax.dot_general(nn_n, anc_n, (((1,), (1,)), ((), ())),
                                    preferred_element_type=jnp.float32)
            ATs.append(jnp.where(diag, 0.0, geo * jnp.maximum(featT, 0.0)))

    vs = [jnp.full((1, K), 1.0 / (K ** 0.5), jnp.float32)] * (2 * G)
    for _ in range(POWER_ITERS):
        us = [jnp.dot(v, AT, preferred_element_type=jnp.float32)
              for v, AT in zip(vs, ATs)]
        vs = [u / (jnp.sqrt(jnp.sum(u * u)) + 1e-8) for u in us]

    rer_rows, sc_rows = [], []
    for g in range(G):
        rer_acc = jnp.zeros((1, 128), jnp.float32)
        sc_acc = jnp.zeros((1, 128), jnp.float32)
        li = lax.broadcasted_iota(jnp.int32, (1, 128), 1)
        for n in range(2):
            v, AT = vs[2 * g + n], ATs[2 * g + n]
            w = jnp.dot(v, AT, preferred_element_type=jnp.float32)
            sc_val = jnp.sum(w * v)

            v_colv = _col(v, diag)
            gt = (v > v_colv).astype(jnp.float32)
            tie = ((v == v_colv) & (ci < ri)).astype(jnp.float32)
            r = jnp.sum(gt + tie, axis=1, keepdims=True)
            Q = (r == ci.astype(jnp.float32)).astype(jnp.float32)
            sorted_v = jnp.dot(v, Q, preferred_element_type=jnp.float32)

            h = jnp.maximum(
                jnp.dot(sorted_v, W1_ref[...],
                        preferred_element_type=jnp.float32) + b1_ref[...], 0.0)
            o = jnp.sum(h * w2t_ref[...]) + b2_ref[0, 0]
            rer_acc = rer_acc + jnp.where(li == n,
                                          1.0 / (1.0 + jnp.exp(-o)), 0.0)
            sc_acc = sc_acc + jnp.where(li == n, sc_val, 0.0)
        rer_rows.append(rer_acc)
        sc_rows.append(sc_acc)

    rer_ref[0] = jnp.concatenate(rer_rows, axis=0)
    sc_ref[0] = jnp.concatenate(sc_rows, axis=0)


def _run_sgv(normP, cent_u, anc_indices, pos_indices, neg_indices,
             W1, b1, W2, b2):
    in_specs, p_args, c_args = [], [], []
    sel_of = {0: (lambda a, p, n: a), 1: (lambda a, p, n: p),
              2: (lambda a, p, n: n)}
    for g in range(G):
        for s in range(3):
            in_specs.append(pl.BlockSpec(
                (1, K, C),
                lambda t, a, p, n, g=g, s=s: (sel_of[s](a, p, n)[t * G + g],
                                              0, 0)))
            p_args.append(normP)
    for g in range(G):
        for s in range(3):
            in_specs.append(pl.BlockSpec(
                (1, 8, K),
                lambda t, a, p, n, g=g, s=s: (sel_of[s](a, p, n)[t * G + g],
                                              0, 0)))
            c_args.append(cent_u)
    in_specs += [
        pl.BlockSpec((K, K), lambda t, a, p, n: (0, 0)),
        pl.BlockSpec((1, K), lambda t, a, p, n: (0, 0)),
        pl.BlockSpec((1, K), lambda t, a, p, n: (0, 0)),
        pl.BlockSpec(memory_space=pltpu.SMEM),
    ]
    grid_spec = pltpu.PrefetchScalarGridSpec(
        num_scalar_prefetch=3,
        grid=(T // G,),
        in_specs=in_specs,
        out_specs=[
            pl.BlockSpec((1, G, 128), lambda t, a, p, n: (t, 0, 0)),
            pl.BlockSpec((1, G, 128), lambda t, a, p, n: (t, 0, 0)),
        ],
    )
    rer_pad, sc_pad = pl.pallas_call(
        _sgv_body,
        grid_spec=grid_spec,
        out_shape=[
            jax.ShapeDtypeStruct((T // G, G, 128), jnp.float32),
            jax.ShapeDtypeStruct((T // G, G, 128), jnp.float32),
        ],
    )(anc_indices, pos_indices, neg_indices,
      *p_args, *c_args,
      W1, b1.reshape(1, K), W2.reshape(1, K), b2.reshape(1, 1))
    return (rer_pad.reshape(T, 128)[:, :2],
            sc_pad.reshape(T, 128)[:, :2])


def kernel(rt_cls_attn, rt, rt_centroids, shift_and_scale,
           anc_indices, pos_indices, neg_indices,
           W_in, b_in, W1, b1, W2, b2):
    scores = rt_cls_attn[..., 0]
    rt_k, cent_cm = _run_select_gather(
        scores, rt_centroids.reshape(B, 3 * N), rt.reshape(B * N, C))

    normP, cent_u = _run_proj(rt_k, cent_cm, shift_and_scale, W_in, b_in)
    return _run_sgv(normP, cent_u, anc_indices, pos_indices, neg_indices,
                    W1, b1, W2, b2)
